# Initial kernel scaffold; baseline (speedup 1.0000x reference)
import jax
import jax.numpy as jnp
from jax.experimental import pallas as pl


def kernel(inputs, token_table, position_table):
    raise NotImplementedError("write your pallas kernel here")


# speedup vs baseline: 1.0418x; 1.0418x over previous
import functools
import math

import jax
import jax.numpy as jnp
from jax import lax
from jax.experimental import pallas as pl
from jax.experimental.pallas import tpu as pltpu, tpu_sc as plsc

_B = 4096
_S = 200
_D = 32
_L = 16

_NC = 2
_NS = 16
_NW = _NC * _NS

_ROWS_PER_W = _B * _S // _NW
_CHUNK_BR = 4
_CHUNK = _CHUNK_BR * _S
_NCHUNK = _ROWS_PER_W // _CHUNK

_SCALE = math.sqrt(float(_D))
_INV_SCALE = 1.0 / _SCALE

_GSZ = 128
_NG_FULL = _CHUNK // _GSZ
_GREM = _CHUNK - _NG_FULL * _GSZ


def _body(idx_hbm, tok_hbm, pos_hbm, out_hbm, idxb, inb, outb, posb, gsem):
    wid = lax.axis_index("s") * _NC + lax.axis_index("c")
    base = wid * _ROWS_PER_W

    pltpu.sync_copy(pos_hbm, posb)

    def _prescale(s, _):
        p0 = posb[s, 0:_L]
        posb[s, 0:_L] = p0 * _INV_SCALE
        p1 = posb[s, _L:_D]
        posb[s, _L:_D] = p1 * _INV_SCALE
        return _

    lax.fori_loop(0, _S, _prescale, 0, unroll=2)

    def _chunk(k, _):
        g0 = base + k * _CHUNK
        pltpu.sync_copy(idx_hbm.at[pl.ds(g0, _CHUNK)], idxb.at[pl.ds(0, _CHUNK)])

        cps = []
        for g in range(_NG_FULL):
            cps.append(pltpu.async_copy(
                tok_hbm.at[idxb.at[pl.ds(g * _GSZ, _GSZ)]],
                inb.at[pl.ds(g * _GSZ, _GSZ)],
                gsem,
            ))
        cps.append(pltpu.async_copy(
            tok_hbm.at[idxb.at[pl.ds(_NG_FULL * _GSZ, _GREM)]],
            inb.at[pl.ds(_NG_FULL * _GSZ, _GREM)],
            gsem,
        ))
        for cp in cps:
            cp.wait()

        def _row(s, _):
            p0 = posb[s, 0:_L]
            p1 = posb[s, _L:_D]
            for br in range(_CHUNK_BR):
                r = br * _S + s
                ivv = idxb[pl.ds(r, _L)]
                m = jnp.where(ivv[0] != 0, _SCALE, 0.0)
                t0 = inb[r, 0:_L]
                outb[r, 0:_L] = (t0 + p0) * m
                t1 = inb[r, _L:_D]
                outb[r, _L:_D] = (t1 + p1) * m
            return _

        lax.fori_loop(0, _S, _row, 0)

        pltpu.sync_copy(outb, out_hbm.at[pl.ds(g0, _CHUNK)])
        return _

    lax.fori_loop(0, _NCHUNK, _chunk, 0)


_mesh = plsc.VectorSubcoreMesh(core_axis_name="c", subcore_axis_name="s")

_kern = functools.partial(
    pl.kernel,
    out_type=jax.ShapeDtypeStruct((_B * _S, _D), jnp.float32),
    mesh=_mesh,
    scratch_types=[
        pltpu.VMEM((_CHUNK + _L,), jnp.int32),
        pltpu.VMEM((_CHUNK, _D), jnp.float32),
        pltpu.VMEM((_CHUNK, _D), jnp.float32),
        pltpu.VMEM((_S, _D), jnp.float32),
        pltpu.SemaphoreType.DMA,
    ],
    compiler_params=pltpu.CompilerParams(use_tc_tiling_on_sc=False),
)(_body)


@jax.jit
def kernel(inputs, token_table, position_table):
    idx_flat = inputs.reshape(-1)
    out = _kern(idx_flat, token_table, position_table)
    return out.reshape(_B, _S, _D)


# speedup vs baseline: 1.1780x; 1.1307x over previous
import functools
import math

import jax
import jax.numpy as jnp
from jax import lax
from jax.experimental import pallas as pl
from jax.experimental.pallas import tpu as pltpu, tpu_sc as plsc

_B = 4096
_S = 200
_D = 32
_L = 16

_NC = 2
_NS = 16
_NW = _NC * _NS

_BPW = _B // _NW
_NBG = _BPW // _L

_SCALE = math.sqrt(float(_D))
_INV_SCALE = 1.0 / _SCALE


def _body(idxT_hbm, idxQ_hbm, tok_hbm, pos_hbm, out_hbm,
          sidx, qidx, inb, outb, posb,
          gsem0, gsem1, wsem0, wsem1,
          isem0, isem1, isem2, isem3, qsem0, qsem1, qsem2, qsem3):
    wid = lax.axis_index("s") * _NC + lax.axis_index("c")
    gsems = (gsem0, gsem1)
    wsems = (wsem0, wsem1)
    isems = (isem0, isem1, isem2, isem3)
    qsems = (qsem0, qsem1, qsem2, qsem3)

    pltpu.sync_copy(pos_hbm, posb)

    def _ps(i, carry):
        for j in range(8):
            v = posb[i, pl.ds(j * _L, _L)]
            posb[i, pl.ds(j * _L, _L)] = v * _INV_SCALE
        return carry

    lax.fori_loop(0, _S * _D // 128, _ps, 0)

    def _fire_idx(s, si):
        pltpu.async_copy(idxT_hbm.at[s, wid], sidx.at[si], isems[si])
        pltpu.async_copy(idxQ_hbm.at[s, wid], qidx.at[si], qsems[si])

    def _drain_idx(s, si):
        pltpu.make_async_copy(
            idxT_hbm.at[s, wid], sidx.at[si], isems[si]).wait()
        pltpu.make_async_copy(
            idxQ_hbm.at[s, wid], qidx.at[si], qsems[si]).wait()

    def _fire_gather(si, sb):
        pltpu.async_copy(tok_hbm.at[qidx.at[si]], inb.at[sb], gsems[sb])

    def _drain_gather(si, sb):
        pltpu.make_async_copy(
            tok_hbm.at[qidx.at[si]], inb.at[sb], gsems[sb]).wait()

    def _drain_write(sb):
        pltpu.make_async_copy(
            outb.at[sb], out_hbm.at[pl.ds(0, 4)], wsems[sb]).wait()

    pltpu.sync_copy(idxT_hbm.at[0, wid], sidx.at[0])
    pltpu.sync_copy(idxQ_hbm.at[0, wid], qidx.at[0])
    pltpu.sync_copy(idxT_hbm.at[1, wid], sidx.at[1])
    pltpu.sync_copy(idxQ_hbm.at[1, wid], qidx.at[1])
    _fire_gather(0, 0)

    def _step(s, s4, j):
        sb = j % 2
        si = j
        si1 = (j + 1) % 4
        si2 = (j + 2) % 4
        _drain_gather(si, sb)

        @pl.when(s + 2 < _S)
        def _():
            _fire_idx(s + 2, si2)

        @pl.when(jnp.logical_and(s >= 1, s + 1 < _S))
        def _():
            _drain_idx(s + 1, si1)

        @pl.when(s + 1 < _S)
        def _():
            _fire_gather(si1, 1 - sb)

        @pl.when(s >= 2)
        def _():
            _drain_write(sb)

        pp0 = posb[s4, pl.ds(j * _D, _L)]
        pp1 = posb[s4, pl.ds(j * _D + _L, _L)]
        iota = lax.iota(jnp.int32, _L)

        def _bg(bg, carry):
            ivv = sidx[si, pl.ds(bg * _L, _L)]
            mv = jnp.where(ivv != 0, _SCALE, 0.0)
            coff = (ivv & 3) << 5
            bvec = iota + bg * _L
            for c in range(_D):
                ppc = pp0[c] if c < _L else pp1[c - _L]
                t = plsc.load_gather(inb.at[sb], [bvec, coff + c])
                outb[sb, c // 8, c % 8, pl.ds(bg * _L, _L)] = (t + ppc) * mv
            return carry

        lax.fori_loop(0, _NBG, _bg, 0)

        for ct in range(4):
            pltpu.async_copy(
                outb.at[sb, ct],
                out_hbm.at[(s * 4 + ct) * _NW + wid],
                wsems[sb],
            )

    def _s4loop(s4, carry):
        for j in range(4):
            _step(s4 * 4 + j, s4, j)
        return carry

    lax.fori_loop(0, _S // 4, _s4loop, 0)

    _drain_write(0)
    _drain_write(1)


_mesh = plsc.VectorSubcoreMesh(core_axis_name="c", subcore_axis_name="s")

_kern = functools.partial(
    pl.kernel,
    out_type=jax.ShapeDtypeStruct((_S * 4 * _NW, 8, 128), jnp.float32),
    mesh=_mesh,
    scratch_types=[
        pltpu.VMEM((4, _BPW), jnp.int32),
        pltpu.VMEM((4, _BPW), jnp.int32),
        pltpu.VMEM((2, _BPW, 128), jnp.float32),
        pltpu.VMEM((2, 4, 8, 128), jnp.float32),
        pltpu.VMEM((_S * _D // 128, 128), jnp.float32),
        pltpu.SemaphoreType.DMA,
        pltpu.SemaphoreType.DMA,
        pltpu.SemaphoreType.DMA,
        pltpu.SemaphoreType.DMA,
        pltpu.SemaphoreType.DMA,
        pltpu.SemaphoreType.DMA,
        pltpu.SemaphoreType.DMA,
        pltpu.SemaphoreType.DMA,
        pltpu.SemaphoreType.DMA,
        pltpu.SemaphoreType.DMA,
        pltpu.SemaphoreType.DMA,
        pltpu.SemaphoreType.DMA,
    ],
    compiler_params=pltpu.CompilerParams(
        use_tc_tiling_on_sc=False, needs_layout_passes=False),
)(_body)


@jax.jit
def kernel(inputs, token_table, position_table):
    idxT = inputs.transpose(1, 0).reshape(_S, _NW, _BPW)
    idxQ = jnp.right_shift(idxT, 2)
    tok2 = token_table.reshape(-1, 128)
    pos2 = position_table.reshape(_S * _D // 128, 128)
    out = _kern(idxT, idxQ, tok2, pos2)
    out = out.reshape(_S, 4, _NW, 8, 128)
    out = out.transpose(2, 4, 0, 1, 3)
    return out.reshape(_B, _S, _D)


# speedup vs baseline: 1.6864x; 1.4316x over previous
import functools
import math

import jax
import jax.numpy as jnp
from jax import lax
from jax.experimental import pallas as pl
from jax.experimental.pallas import tpu as pltpu, tpu_sc as plsc

_B = 4096
_S = 200
_D = 32
_L = 16

_NC = 2
_NS = 16
_NW = _NC * _NS

_BPW = _B // _NW
_NBG = _BPW // _L

_SCALE = math.sqrt(float(_D))
_INV_SCALE = 1.0 / _SCALE


def _body(idxT_hbm, idxQ_hbm, tok_hbm, pos_hbm, out_hbm,
          sidx, qidx, inb, outb, posb,
          gsem0, gsem1, wsem0, wsem1,
          isem0, isem1, isem2, isem3, qsem0, qsem1, qsem2, qsem3):
    wid = lax.axis_index("s") * _NC + lax.axis_index("c")
    gsems = (gsem0, gsem1)
    wsems = (wsem0, wsem1)
    isems = (isem0, isem1, isem2, isem3)
    qsems = (qsem0, qsem1, qsem2, qsem3)

    pltpu.sync_copy(pos_hbm, posb)

    def _ps(i, carry):
        for j in range(8):
            v = posb[i, pl.ds(j * _L, _L)]
            posb[i, pl.ds(j * _L, _L)] = v * _INV_SCALE
        return carry

    lax.fori_loop(0, _S * _D // 128, _ps, 0)

    def _fire_idx(s, si):
        pltpu.async_copy(idxT_hbm.at[s, wid], sidx.at[si], isems[si])
        pltpu.async_copy(idxQ_hbm.at[s, wid], qidx.at[si], qsems[si])

    def _drain_idx(s, si):
        pltpu.make_async_copy(
            idxT_hbm.at[s, wid], sidx.at[si], isems[si]).wait()
        pltpu.make_async_copy(
            idxQ_hbm.at[s, wid], qidx.at[si], qsems[si]).wait()

    def _fire_gather(si, sb):
        pltpu.async_copy(tok_hbm.at[qidx.at[si]], inb.at[sb], gsems[sb])

    def _drain_gather(si, sb):
        pltpu.make_async_copy(
            tok_hbm.at[qidx.at[si]], inb.at[sb], gsems[sb]).wait()

    def _drain_write(sb):
        pltpu.make_async_copy(
            outb.at[sb], out_hbm.at[pl.ds(0, 4)], wsems[sb]).wait()

    pltpu.sync_copy(idxT_hbm.at[0, wid], sidx.at[0])
    pltpu.sync_copy(idxQ_hbm.at[0, wid], qidx.at[0])
    pltpu.sync_copy(idxT_hbm.at[1, wid], sidx.at[1])
    pltpu.sync_copy(idxQ_hbm.at[1, wid], qidx.at[1])
    _fire_gather(0, 0)

    def _step(s, s4, j):
        sb = j % 2
        si = j
        si1 = (j + 1) % 4
        si2 = (j + 2) % 4
        _drain_gather(si, sb)

        @pl.when(s + 2 < _S)
        def _():
            _fire_idx(s + 2, si2)

        @pl.when(jnp.logical_and(s >= 1, s + 1 < _S))
        def _():
            _drain_idx(s + 1, si1)

        @pl.when(s + 1 < _S)
        def _():
            _fire_gather(si1, 1 - sb)

        @pl.when(s >= 2)
        def _():
            _drain_write(sb)

        pp0 = posb[s4, pl.ds(j * _D, _L)]
        pp1 = posb[s4, pl.ds(j * _D + _L, _L)]
        iota = lax.iota(jnp.int32, _L)

        def _bg(bg, carry):
            ivv = sidx[si, pl.ds(bg * _L, _L)]
            mv = jnp.where(ivv != 0, _SCALE, 0.0)
            coff = (ivv & 3) << 5
            bvec = iota + bg * _L
            ts = [plsc.load_gather(inb.at[sb], [bvec, coff + c])
                  for c in range(_D)]
            for c in range(_D):
                ppc = pp0[c] if c < _L else pp1[c - _L]
                outb[sb, c // 8, c % 8, pl.ds(bg * _L, _L)] = (ts[c] + ppc) * mv
            return carry

        lax.fori_loop(0, _NBG, _bg, 0)

        for ct in range(4):
            pltpu.async_copy(
                outb.at[sb, ct],
                out_hbm.at[(s * 4 + ct) * _NW + wid],
                wsems[sb],
            )

    def _s4loop(s4, carry):
        for j in range(4):
            _step(s4 * 4 + j, s4, j)
        return carry

    lax.fori_loop(0, _S // 4, _s4loop, 0)

    _drain_write(0)
    _drain_write(1)


_mesh = plsc.VectorSubcoreMesh(core_axis_name="c", subcore_axis_name="s")

_kern = functools.partial(
    pl.kernel,
    out_type=jax.ShapeDtypeStruct((_S * 4 * _NW, 8, 128), jnp.float32),
    mesh=_mesh,
    scratch_types=[
        pltpu.VMEM((4, _BPW), jnp.int32),
        pltpu.VMEM((4, _BPW), jnp.int32),
        pltpu.VMEM((2, _BPW, 128), jnp.float32),
        pltpu.VMEM((2, 4, 8, 128), jnp.float32),
        pltpu.VMEM((_S * _D // 128, 128), jnp.float32),
        pltpu.SemaphoreType.DMA,
        pltpu.SemaphoreType.DMA,
        pltpu.SemaphoreType.DMA,
        pltpu.SemaphoreType.DMA,
        pltpu.SemaphoreType.DMA,
        pltpu.SemaphoreType.DMA,
        pltpu.SemaphoreType.DMA,
        pltpu.SemaphoreType.DMA,
        pltpu.SemaphoreType.DMA,
        pltpu.SemaphoreType.DMA,
        pltpu.SemaphoreType.DMA,
        pltpu.SemaphoreType.DMA,
    ],
    compiler_params=pltpu.CompilerParams(
        use_tc_tiling_on_sc=False, needs_layout_passes=False),
)(_body)


@jax.jit
def kernel(inputs, token_table, position_table):
    idxT = inputs.transpose(1, 0).reshape(_S, _NW, _BPW)
    idxQ = jnp.right_shift(idxT, 2)
    tok2 = token_table.reshape(-1, 128)
    pos2 = position_table.reshape(_S * _D // 128, 128)
    out = _kern(idxT, idxQ, tok2, pos2)
    out = out.reshape(_S, 4, _NW, 8, 128)
    out = out.transpose(2, 4, 0, 1, 3)
    return out.reshape(_B, _S, _D)


# speedup vs baseline: 1.7136x; 1.0161x over previous
import functools
import math

import jax
import jax.numpy as jnp
from jax import lax
from jax.experimental import pallas as pl
from jax.experimental.pallas import tpu as pltpu, tpu_sc as plsc

_B = 4096
_S = 200
_D = 32
_L = 16

_NC = 2
_NS = 16
_NW = _NC * _NS

_BPW = _B // _NW
_NBG = _BPW // _L

_SCALE = math.sqrt(float(_D))
_INV_SCALE = 1.0 / _SCALE


def _body(idxT_hbm, tok_hbm, pos_hbm, out_hbm,
          sidx, inb, outb, posb,
          gsem0, gsem1, wsem0, wsem1,
          isem0, isem1, isem2, isem3):
    wid = lax.axis_index("s") * _NC + lax.axis_index("c")
    gsems = (gsem0, gsem1)
    wsems = (wsem0, wsem1)
    isems = (isem0, isem1, isem2, isem3)

    pltpu.sync_copy(pos_hbm, posb)

    def _ps(i, carry):
        for j in range(8):
            v = posb[i, pl.ds(j * _L, _L)]
            posb[i, pl.ds(j * _L, _L)] = v * _INV_SCALE
        return carry

    lax.fori_loop(0, _S * _D // 128, _ps, 0)

    def _fire_idx(s, si):
        pltpu.async_copy(idxT_hbm.at[s, wid], sidx.at[si], isems[si])

    def _drain_idx(s, si):
        pltpu.make_async_copy(
            idxT_hbm.at[s, wid], sidx.at[si], isems[si]).wait()

    def _fire_gather(si, sb):
        pltpu.async_copy(tok_hbm.at[sidx.at[si]], inb.at[sb], gsems[sb])

    def _drain_gather(si, sb):
        pltpu.make_async_copy(
            tok_hbm.at[sidx.at[si]], inb.at[sb], gsems[sb]).wait()

    def _drain_write(sb):
        pltpu.make_async_copy(
            outb.at[sb], out_hbm.at[pl.ds(0, 4)], wsems[sb]).wait()

    pltpu.sync_copy(idxT_hbm.at[0, wid], sidx.at[0])
    pltpu.sync_copy(idxT_hbm.at[1, wid], sidx.at[1])
    _fire_gather(0, 0)

    def _step(s, s4, j):
        sb = j % 2
        si = j
        si1 = (j + 1) % 4
        si2 = (j + 2) % 4
        _drain_gather(si, sb)

        @pl.when(s + 2 < _S)
        def _():
            _fire_idx(s + 2, si2)

        @pl.when(jnp.logical_and(s >= 1, s + 1 < _S))
        def _():
            _drain_idx(s + 1, si1)

        @pl.when(s + 1 < _S)
        def _():
            _fire_gather(si1, 1 - sb)

        @pl.when(s >= 2)
        def _():
            _drain_write(sb)

        pp0 = posb[s4, pl.ds(j * _D, _L)]
        pp1 = posb[s4, pl.ds(j * _D + _L, _L)]
        iota = lax.iota(jnp.int32, _L)

        def _bg(bg, carry):
            ivv = sidx[si, pl.ds(bg * _L, _L)]
            mv = jnp.where(ivv != 0, _SCALE, 0.0)
            bvec = iota + bg * _L
            ts = [plsc.load_gather(
                      inb.at[sb],
                      [bvec, jnp.full((_L,), c, dtype=jnp.int32)])
                  for c in range(_D)]
            for c in range(_D):
                ppc = pp0[c] if c < _L else pp1[c - _L]
                outb[sb, c // 8, c % 8, pl.ds(bg * _L, _L)] = (ts[c] + ppc) * mv
            return carry

        lax.fori_loop(0, _NBG, _bg, 0)

        for ct in range(4):
            pltpu.async_copy(
                outb.at[sb, ct],
                out_hbm.at[(s * 4 + ct) * _NW + wid],
                wsems[sb],
            )

    def _s4loop(s4, carry):
        for j in range(4):
            _step(s4 * 4 + j, s4, j)
        return carry

    lax.fori_loop(0, _S // 4, _s4loop, 0)

    _drain_write(0)
    _drain_write(1)


_mesh = plsc.VectorSubcoreMesh(core_axis_name="c", subcore_axis_name="s")

_kern = functools.partial(
    pl.kernel,
    out_type=jax.ShapeDtypeStruct((_S * 4 * _NW, 8, 128), jnp.float32),
    mesh=_mesh,
    scratch_types=[
        pltpu.VMEM((4, _BPW), jnp.int32),
        pltpu.VMEM((2, _BPW, _D), jnp.float32),
        pltpu.VMEM((2, 4, 8, 128), jnp.float32),
        pltpu.VMEM((_S * _D // 128, 128), jnp.float32),
        pltpu.SemaphoreType.DMA,
        pltpu.SemaphoreType.DMA,
        pltpu.SemaphoreType.DMA,
        pltpu.SemaphoreType.DMA,
        pltpu.SemaphoreType.DMA,
        pltpu.SemaphoreType.DMA,
        pltpu.SemaphoreType.DMA,
        pltpu.SemaphoreType.DMA,
    ],
    compiler_params=pltpu.CompilerParams(
        use_tc_tiling_on_sc=False, needs_layout_passes=False),
)(_body)


@jax.jit
def kernel(inputs, token_table, position_table):
    idxT = inputs.transpose(1, 0).reshape(_S, _NW, _BPW)
    pos2 = position_table.reshape(_S * _D // 128, 128)
    out = _kern(idxT, token_table, pos2)
    out = out.reshape(_S, 4, _NW, 8, 128)
    out = out.transpose(2, 4, 0, 1, 3)
    return out.reshape(_B, _S, _D)


# speedup vs baseline: 1.7151x; 1.0009x over previous
import functools
import math

import jax
import jax.numpy as jnp
from jax import lax
from jax.experimental import pallas as pl
from jax.experimental.pallas import tpu as pltpu, tpu_sc as plsc

_B = 4096
_S = 200
_D = 32
_L = 16

_NC = 2
_NS = 16
_NW = _NC * _NS

_BPW = _B // _NW
_NBG = _BPW // _L

_SCALE = math.sqrt(float(_D))
_INV_SCALE = 1.0 / _SCALE


def _body(idxT_hbm, tok_hbm, pos_hbm, out_hbm,
          sidx, inb, outb, posb,
          gsem0, gsem1, gsem2, gsem3, wsem0, wsem1,
          isem0, isem1, isem2, isem3):
    wid = lax.axis_index("s") * _NC + lax.axis_index("c")
    gsems = (gsem0, gsem1, gsem2, gsem3)
    wsems = (wsem0, wsem1)
    isems = (isem0, isem1, isem2, isem3)

    pltpu.sync_copy(pos_hbm, posb)

    def _ps(i, carry):
        for j in range(8):
            v = posb[i, pl.ds(j * _L, _L)]
            posb[i, pl.ds(j * _L, _L)] = v * _INV_SCALE
        return carry

    lax.fori_loop(0, _S * _D // 128, _ps, 0)

    def _fire_idx(s, si):
        pltpu.async_copy(idxT_hbm.at[s, wid], sidx.at[si], isems[si])

    def _drain_idx(s, si):
        pltpu.make_async_copy(
            idxT_hbm.at[s, wid], sidx.at[si], isems[si]).wait()

    def _fire_gather(si):
        pltpu.async_copy(tok_hbm.at[sidx.at[si]], inb.at[si], gsems[si])

    def _drain_gather(si):
        pltpu.make_async_copy(
            tok_hbm.at[sidx.at[si]], inb.at[si], gsems[si]).wait()

    def _drain_write(sb):
        pltpu.make_async_copy(
            outb.at[sb], out_hbm.at[pl.ds(0, 4)], wsems[sb]).wait()

    pltpu.sync_copy(idxT_hbm.at[0, wid], sidx.at[0])
    pltpu.sync_copy(idxT_hbm.at[1, wid], sidx.at[1])
    _fire_gather(0)
    _fire_gather(1)
    _fire_idx(2, 2)

    def _step(s, s4, j):
        sb = j % 2
        si = j
        si2 = (j + 2) % 4
        si3 = (j + 3) % 4
        _drain_gather(si)

        @pl.when(s + 3 < _S)
        def _():
            _fire_idx(s + 3, si3)

        @pl.when(s + 2 < _S)
        def _():
            _drain_idx(s + 2, si2)

        @pl.when(s + 2 < _S)
        def _():
            _fire_gather(si2)

        @pl.when(s >= 2)
        def _():
            _drain_write(sb)

        pp0 = posb[s4, pl.ds(j * _D, _L)]
        pp1 = posb[s4, pl.ds(j * _D + _L, _L)]
        iota = lax.iota(jnp.int32, _L)

        def _bg(bg, carry):
            ivv = sidx[si, pl.ds(bg * _L, _L)]
            mv = jnp.where(ivv != 0, _SCALE, 0.0)
            bvec = iota + bg * _L
            ts = [plsc.load_gather(
                      inb.at[si],
                      [bvec, jnp.full((_L,), c, dtype=jnp.int32)])
                  for c in range(_D)]
            for c in range(_D):
                ppc = pp0[c] if c < _L else pp1[c - _L]
                outb[sb, c // 8, c % 8, pl.ds(bg * _L, _L)] = (ts[c] + ppc) * mv
            return carry

        lax.fori_loop(0, _NBG, _bg, 0)

        for ct in range(4):
            pltpu.async_copy(
                outb.at[sb, ct],
                out_hbm.at[(s * 4 + ct) * _NW + wid],
                wsems[sb],
            )

    def _s4loop(s4, carry):
        for j in range(4):
            _step(s4 * 4 + j, s4, j)
        return carry

    lax.fori_loop(0, _S // 4, _s4loop, 0)

    _drain_write(0)
    _drain_write(1)


_mesh = plsc.VectorSubcoreMesh(core_axis_name="c", subcore_axis_name="s")

_kern = functools.partial(
    pl.kernel,
    out_type=jax.ShapeDtypeStruct((_S * 4 * _NW, 8, 128), jnp.float32),
    mesh=_mesh,
    scratch_types=[
        pltpu.VMEM((4, _BPW), jnp.int32),
        pltpu.VMEM((4, _BPW, _D), jnp.float32),
        pltpu.VMEM((2, 4, 8, 128), jnp.float32),
        pltpu.VMEM((_S * _D // 128, 128), jnp.float32),
        pltpu.SemaphoreType.DMA,
        pltpu.SemaphoreType.DMA,
        pltpu.SemaphoreType.DMA,
        pltpu.SemaphoreType.DMA,
        pltpu.SemaphoreType.DMA,
        pltpu.SemaphoreType.DMA,
        pltpu.SemaphoreType.DMA,
        pltpu.SemaphoreType.DMA,
        pltpu.SemaphoreType.DMA,
        pltpu.SemaphoreType.DMA,
    ],
    compiler_params=pltpu.CompilerParams(
        use_tc_tiling_on_sc=False, needs_layout_passes=False),
)(_body)


@jax.jit
def kernel(inputs, token_table, position_table):
    idxT = inputs.transpose(1, 0).reshape(_S, _NW, _BPW)
    pos2 = position_table.reshape(_S * _D // 128, 128)
    out = _kern(idxT, token_table, pos2)
    out = out.reshape(_S, 4, _NW, 8, 128)
    out = out.transpose(2, 4, 0, 1, 3)
    return out.reshape(_B, _S, _D)
